# baseline (speedup 1.0000x reference)
import jax
import jax.numpy as jnp
from jax import lax
from jax.experimental import pallas as pl
from jax.experimental.pallas import tpu as pltpu
from jax.experimental.pallas import tpu_sc as plsc

N = 10000
E = 320000
IN_CH = 128
HID = 64
NG = 64

NC, NS, L = 2, 16, 16
NW = NC * NS
CH = 128

RPT = 632
NPADR = RPT * NS
ZRT = NPADR // NS
CNTW = 16

EPW = E // NW
K = -(-EPW // CH)
PAD = K * CH - EPW

EPT = E // NS
KH = -(-EPT // (2 * CH))
PADA = NS * 2 * KH * CH - E


def _pad_dst(n_pad):
  return N + (jnp.arange(n_pad, dtype=jnp.int32) % (NPADR - N))


def _sc_pass_alledges():
  out_type = [jax.ShapeDtypeStruct((NC * NPADR, HID), jnp.float32),
              jax.ShapeDtypeStruct((NPADR, CNTW), jnp.float32)]
  scratch = {
      "srcv": pltpu.VMEM((KH, CH), jnp.int32),
      "dstv": pltpu.VMEM((KH, CH), jnp.int32),
      "buf": pltpu.VMEM((CH, HID), jnp.float32),
      "onesv": pltpu.VMEM((CH, CNTW), jnp.float32),
      "acc": pltpu.VMEM_SHARED((NPADR, HID), jnp.float32),
      "cntacc": pltpu.VMEM_SHARED((NPADR, CNTW), jnp.float32),
      "zsp": pltpu.VMEM_SHARED((NPADR, HID), jnp.float32),
      "sem": pltpu.SemaphoreType.DMA,
  }
  mesh = plsc.VectorSubcoreMesh(
      core_axis_name="c", subcore_axis_name="s",
      num_cores=NC, num_subcores=NS)

  def body(xs, srcs, dsts, zrow, zcnt, ones, out, cntout, *, srcv, dstv,
           buf, onesv, acc, cntacc, zsp, sem):
    cid = lax.axis_index("c")
    sid = lax.axis_index("s")
    pltpu.sync_copy(xs.at[cid, pl.ds(sid * ZRT, ZRT)],
                    zsp.at[pl.ds(sid * ZRT, ZRT)])
    pltpu.sync_copy(zrow, acc.at[pl.ds(sid * RPT, RPT)])
    pltpu.sync_copy(ones, onesv)

    @pl.when(cid == 0)
    def _():
      pltpu.sync_copy(zcnt, cntacc.at[pl.ds(sid * RPT, RPT)])
    plsc.subcore_barrier()

    for s in range(2):
      pltpu.sync_copy(srcs.at[sid, s], srcv)
      pltpu.sync_copy(dsts.at[sid, s], dstv)

      def step(j, carry):
        pltpu.async_copy(zsp.at[srcv.at[j]], buf, sem).wait()
        pltpu.sync_copy(buf, acc.at[dstv.at[j]], add=True)

        @pl.when(cid == 0)
        def _():
          pltpu.sync_copy(onesv, cntacc.at[dstv.at[j]], add=True)
        return carry
      lax.fori_loop(0, KH, step, 0)

    plsc.subcore_barrier()
    obase = cid * NPADR + sid * RPT
    pltpu.sync_copy(acc.at[pl.ds(sid * RPT, RPT)], out.at[pl.ds(obase, RPT)])

    @pl.when(cid == 0)
    def _():
      pltpu.sync_copy(cntacc.at[pl.ds(sid * RPT, RPT)],
                      cntout.at[pl.ds(sid * RPT, RPT)])

  return pl.kernel(body, out_type=out_type, mesh=mesh,
                   scratch_types=scratch,
                   compiler_params=pltpu.CompilerParams(
                       use_tc_tiling_on_sc=False))


def _sc_pass_split():
  out_type = [jax.ShapeDtypeStruct((NC * NPADR, HID), jnp.float32)]
  scratch = {
      "srcv": pltpu.VMEM((K, CH), jnp.int32),
      "dstv": pltpu.VMEM((K, CH), jnp.int32),
      "buf": pltpu.VMEM((CH, HID), jnp.float32),
      "acc": pltpu.VMEM_SHARED((NPADR, HID), jnp.float32),
      "zsp": pltpu.VMEM_SHARED((NPADR, HID), jnp.float32),
      "sem": pltpu.SemaphoreType.DMA,
  }
  mesh = plsc.VectorSubcoreMesh(
      core_axis_name="c", subcore_axis_name="s",
      num_cores=NC, num_subcores=NS)

  def body(z, srcs, dsts, zrow, out, *, srcv, dstv, buf, acc, zsp, sem):
    cid = lax.axis_index("c")
    sid = lax.axis_index("s")
    wid = cid * NS + sid
    pltpu.sync_copy(srcs.at[wid], srcv)
    pltpu.sync_copy(dsts.at[wid], dstv)
    pltpu.sync_copy(z.at[pl.ds(sid * ZRT, ZRT)],
                    zsp.at[pl.ds(sid * ZRT, ZRT)])
    pltpu.sync_copy(zrow, acc.at[pl.ds(sid * RPT, RPT)])
    plsc.subcore_barrier()

    def step(j, carry):
      pltpu.async_copy(zsp.at[srcv.at[j]], buf, sem).wait()
      pltpu.sync_copy(buf, acc.at[dstv.at[j]], add=True)
      return carry
    lax.fori_loop(0, K, step, 0)

    plsc.subcore_barrier()
    obase = cid * NPADR + sid * RPT
    pltpu.sync_copy(acc.at[pl.ds(sid * RPT, RPT)], out.at[pl.ds(obase, RPT)])

  return pl.kernel(body, out_type=out_type, mesh=mesh,
                   scratch_types=scratch,
                   compiler_params=pltpu.CompilerParams(
                       use_tc_tiling_on_sc=False))


def _deg(cnt_ref):
  cnt = cnt_ref[pl.ds(0, N)]
  return jnp.maximum(jnp.max(cnt, axis=1, keepdims=True), 1.0)


BR = 2000


def _tc_layer1(accA, cnt, x, Wl, Wr, b):
  def body(acc_ref, cnt_ref, x_ref, wl_ref, wr_ref, b_ref, h_ref):
    cntv = cnt_ref[...]
    deg = jnp.maximum(jnp.max(cntv, axis=1, keepdims=True), 1.0)
    mean = jnp.concatenate([acc_ref[0] / deg, acc_ref[1] / deg], axis=1)
    h = (jnp.dot(mean, wl_ref[...], preferred_element_type=jnp.float32)
         + jnp.dot(x_ref[...], wr_ref[...],
                   preferred_element_type=jnp.float32) + b_ref[...])
    h_ref[...] = jnp.maximum(h, 0.0)
  grid = (N // BR,)
  return pl.pallas_call(
      body,
      grid=grid,
      in_specs=[
          pl.BlockSpec((NC, BR, HID), lambda i: (0, i, 0)),
          pl.BlockSpec((BR, CNTW), lambda i: (i, 0)),
          pl.BlockSpec((BR, IN_CH), lambda i: (i, 0)),
          pl.BlockSpec((IN_CH, HID), lambda i: (0, 0)),
          pl.BlockSpec((IN_CH, HID), lambda i: (0, 0)),
          pl.BlockSpec((HID,), lambda i: (0,)),
      ],
      out_specs=pl.BlockSpec((BR, HID), lambda i: (i, 0)),
      out_shape=jax.ShapeDtypeStruct((N, HID), jnp.float32),
  )(accA.reshape(NC, NPADR, HID), cnt, x, Wl, Wr, b)


def _tc_layer2(accp, cnt, h_prev, Wl, Wr, b):
  def body(acc_ref, cnt_ref, hp_ref, wl_ref, wr_ref, b_ref, h_ref):
    cntv = cnt_ref[...]
    deg = jnp.maximum(jnp.max(cntv, axis=1, keepdims=True), 1.0)
    mean = (acc_ref[0] + acc_ref[1]) / deg
    h = (jnp.dot(mean, wl_ref[...], preferred_element_type=jnp.float32)
         + jnp.dot(hp_ref[...], wr_ref[...],
                   preferred_element_type=jnp.float32) + b_ref[...])
    h_ref[...] = jnp.maximum(h, 0.0)
  grid = (N // BR,)
  return pl.pallas_call(
      body,
      grid=grid,
      in_specs=[
          pl.BlockSpec((NC, BR, HID), lambda i: (0, i, 0)),
          pl.BlockSpec((BR, CNTW), lambda i: (i, 0)),
          pl.BlockSpec((BR, HID), lambda i: (i, 0)),
          pl.BlockSpec((HID, HID), lambda i: (0, 0)),
          pl.BlockSpec((HID, HID), lambda i: (0, 0)),
          pl.BlockSpec((HID,), lambda i: (0,)),
      ],
      out_specs=pl.BlockSpec((BR, HID), lambda i: (i, 0)),
      out_shape=jax.ShapeDtypeStruct((N, HID), jnp.float32),
  )(accp.reshape(NC, NPADR, HID), cnt, h_prev, Wl, Wr, b)


def _tc_final(accp, cnt, h_prev, batch2d, Wl, Wr, b, Wc1, bc1, Wc2, bc2):
  def body(acc_ref, cnt_ref, hp_ref, b2d_ref, wl_ref, wr_ref, b_ref,
           wc1_ref, bc1_ref, wc2_ref, bc2_ref, out_ref):
    deg = _deg(cnt_ref)
    mean = (acc_ref[pl.ds(0, N)] + acc_ref[pl.ds(NPADR, N)]) / deg
    h = (jnp.dot(mean, wl_ref[...], preferred_element_type=jnp.float32)
         + jnp.dot(hp_ref[...], wr_ref[...],
                   preferred_element_type=jnp.float32) + b_ref[...])
    gid = b2d_ref[...]
    onehot = (gid == lax.broadcasted_iota(jnp.int32, (1, NG), 1))
    onehot = onehot.astype(jnp.float32)
    g = lax.dot_general(onehot, h, (((0,), (0,)), ((), ())),
                        preferred_element_type=jnp.float32,
                        precision=lax.Precision.HIGHEST)
    g = jnp.maximum(
        jnp.dot(g, wc1_ref[...], preferred_element_type=jnp.float32)
        + bc1_ref[...], 0.0)
    out_ref[...] = jnp.dot(
        g, wc2_ref[...], preferred_element_type=jnp.float32) + bc2_ref[...]
  outs = jax.ShapeDtypeStruct((NG, 1), jnp.float32)
  return pl.pallas_call(body, out_shape=outs)(
      accp, cnt, h_prev, batch2d, Wl, Wr, b, Wc1, bc1, Wc2, bc2)


def kernel(x, edge_index, batch, Wl1, Wr1, b1, Wl2, Wr2, b2, Wl3, Wr3, b3,
           Wc1, bc1, Wc2, bc2):
  src = edge_index[0].astype(jnp.int32)
  dst = edge_index[1].astype(jnp.int32)

  srcA = jnp.concatenate([src, jnp.zeros((PADA,), jnp.int32)])
  dstA = jnp.concatenate([dst, _pad_dst(PADA)])
  srcA = srcA.reshape(NS, 2, KH, CH)
  dstA = dstA.reshape(NS, 2, KH, CH)

  padsrc = jnp.zeros((NW, PAD), jnp.int32)
  paddst = jnp.broadcast_to(_pad_dst(PAD), (NW, PAD))
  srcs = jnp.concatenate([src.reshape(NW, EPW), padsrc], 1).reshape(NW, K, CH)
  dsts = jnp.concatenate([dst.reshape(NW, EPW), paddst], 1).reshape(NW, K, CH)

  xpad = jnp.concatenate([x, jnp.zeros((NPADR - N, IN_CH), jnp.float32)])
  xs = jnp.stack([xpad[:, :HID], xpad[:, HID:]])

  zrow = jnp.zeros((RPT, HID), jnp.float32)
  zcnt = jnp.zeros((RPT, CNTW), jnp.float32)
  ones = jnp.ones((CH, CNTW), jnp.float32)
  batch2d = batch.astype(jnp.int32).reshape(N, 1)

  zpadrows = jnp.zeros((NPADR - N, HID), jnp.float32)

  accA, cnt = _sc_pass_alledges()(xs, srcA, dstA, zrow, zcnt, ones)
  h1 = _tc_layer1(accA, cnt, x, Wl1, Wr1, b1)
  h1p = jnp.concatenate([h1, zpadrows])
  (acc2,) = _sc_pass_split()(h1p, srcs, dsts, zrow)
  h2 = _tc_layer2(acc2, cnt, h1, Wl2, Wr2, b2)
  h2p = jnp.concatenate([h2, zpadrows])
  (acc3,) = _sc_pass_split()(h2p, srcs, dsts, zrow)
  return _tc_final(acc3, cnt, h2, batch2d, Wl3, Wr3, b3, Wc1, bc1, Wc2, bc2)


# speedup vs baseline: 1.1104x; 1.1104x over previous
import jax
import jax.numpy as jnp
from jax import lax
from jax.experimental import pallas as pl
from jax.experimental.pallas import tpu as pltpu
from jax.experimental.pallas import tpu_sc as plsc

N = 10000
E = 320000
IN_CH = 128
HID = 64
NG = 64

NC, NS, L = 2, 16, 16
NW = NC * NS
CH = 128

RPT = 632
NPADR = RPT * NS
ZRT = NPADR // NS
CNTW = 16

EPW = E // NW
K = -(-EPW // CH)
PAD = K * CH - EPW

EPT = E // NS
KH = -(-EPT // (2 * CH))
PADA = NS * 2 * KH * CH - E


def _pad_dst(n_pad):
  return N + (jnp.arange(n_pad, dtype=jnp.int32) % (NPADR - N))


def _sc_pass_alledges():
  out_type = [jax.ShapeDtypeStruct((NC * NPADR, HID), jnp.float32),
              jax.ShapeDtypeStruct((NPADR, CNTW), jnp.float32)]
  scratch = {
      "srcv": pltpu.VMEM((KH, CH), jnp.int32),
      "dstv": pltpu.VMEM((KH, CH), jnp.int32),
      "buf": pltpu.VMEM((CH, HID), jnp.float32),
      "onesv": pltpu.VMEM((CH, CNTW), jnp.float32),
      "acc": pltpu.VMEM_SHARED((NPADR, HID), jnp.float32),
      "cntacc": pltpu.VMEM_SHARED((NPADR, CNTW), jnp.float32),
      "zsp": pltpu.VMEM_SHARED((NPADR, HID), jnp.float32),
      "sem": pltpu.SemaphoreType.DMA,
  }
  mesh = plsc.VectorSubcoreMesh(
      core_axis_name="c", subcore_axis_name="s",
      num_cores=NC, num_subcores=NS)

  def body(xs, srcs, dsts, zrow, zcnt, ones, out, cntout, *, srcv, dstv,
           buf, onesv, acc, cntacc, zsp, sem):
    cid = lax.axis_index("c")
    sid = lax.axis_index("s")
    pltpu.sync_copy(xs.at[cid, pl.ds(sid * ZRT, ZRT)],
                    zsp.at[pl.ds(sid * ZRT, ZRT)])
    pltpu.sync_copy(zrow, acc.at[pl.ds(sid * RPT, RPT)])
    pltpu.sync_copy(ones, onesv)

    @pl.when(cid == 0)
    def _():
      pltpu.sync_copy(zcnt, cntacc.at[pl.ds(sid * RPT, RPT)])
    plsc.subcore_barrier()

    for s in range(2):
      pltpu.sync_copy(srcs.at[sid, s], srcv)
      pltpu.sync_copy(dsts.at[sid, s], dstv)

      def step(j, carry):
        pltpu.async_copy(zsp.at[srcv.at[j]], buf, sem).wait()
        pltpu.sync_copy(buf, acc.at[dstv.at[j]], add=True)

        @pl.when(cid == 0)
        def _():
          pltpu.sync_copy(onesv, cntacc.at[dstv.at[j]], add=True)
        return carry
      lax.fori_loop(0, KH, step, 0)

    plsc.subcore_barrier()
    obase = cid * NPADR + sid * RPT
    pltpu.sync_copy(acc.at[pl.ds(sid * RPT, RPT)], out.at[pl.ds(obase, RPT)])

    @pl.when(cid == 0)
    def _():
      pltpu.sync_copy(cntacc.at[pl.ds(sid * RPT, RPT)],
                      cntout.at[pl.ds(sid * RPT, RPT)])

  return pl.kernel(body, out_type=out_type, mesh=mesh,
                   scratch_types=scratch,
                   compiler_params=pltpu.CompilerParams(
                       use_tc_tiling_on_sc=False))


def _sc_pass_split():
  out_type = [jax.ShapeDtypeStruct((NC * NPADR, HID), jnp.float32)]
  scratch = {
      "srcv": pltpu.VMEM((K, CH), jnp.int32),
      "dstv": pltpu.VMEM((K, CH), jnp.int32),
      "buf": pltpu.VMEM((2, CH, HID), jnp.float32),
      "acc": pltpu.VMEM_SHARED((NPADR, HID), jnp.float32),
      "zsp": pltpu.VMEM_SHARED((NPADR, HID), jnp.float32),
      "sem": pltpu.SemaphoreType.DMA,
  }
  mesh = plsc.VectorSubcoreMesh(
      core_axis_name="c", subcore_axis_name="s",
      num_cores=NC, num_subcores=NS)

  def body(z, srcs, dsts, zrow, out, *, srcv, dstv, buf, acc, zsp, sem):
    cid = lax.axis_index("c")
    sid = lax.axis_index("s")
    wid = cid * NS + sid
    pltpu.sync_copy(srcs.at[wid], srcv)
    pltpu.sync_copy(dsts.at[wid], dstv)
    pltpu.sync_copy(z.at[pl.ds(sid * ZRT, ZRT)],
                    zsp.at[pl.ds(sid * ZRT, ZRT)])
    pltpu.sync_copy(zrow, acc.at[pl.ds(sid * RPT, RPT)])
    plsc.subcore_barrier()

    pltpu.async_copy(zsp.at[srcv.at[0]], buf.at[0], sem)

    def step(j, carry):
      p = lax.rem(j, 2)

      @pl.when(j + 1 < K)
      def _():
        pltpu.async_copy(zsp.at[srcv.at[j + 1]], buf.at[1 - p], sem)
      pltpu.make_async_copy(zsp.at[srcv.at[j]], buf.at[p], sem).wait()
      pltpu.sync_copy(buf.at[p], acc.at[dstv.at[j]], add=True)
      return carry
    lax.fori_loop(0, K, step, 0)

    plsc.subcore_barrier()
    obase = cid * NPADR + sid * RPT
    pltpu.sync_copy(acc.at[pl.ds(sid * RPT, RPT)], out.at[pl.ds(obase, RPT)])

  return pl.kernel(body, out_type=out_type, mesh=mesh,
                   scratch_types=scratch,
                   compiler_params=pltpu.CompilerParams(
                       use_tc_tiling_on_sc=False))


def _deg(cnt_ref):
  cnt = cnt_ref[pl.ds(0, N)]
  return jnp.maximum(jnp.max(cnt, axis=1, keepdims=True), 1.0)


BR = 2000


def _tc_layer1(accA, cnt, x, Wl, Wr, b):
  def body(acc_ref, cnt_ref, x_ref, wl_ref, wr_ref, b_ref, h_ref):
    cntv = cnt_ref[...]
    deg = jnp.maximum(jnp.max(cntv, axis=1, keepdims=True), 1.0)
    mean = jnp.concatenate([acc_ref[0] / deg, acc_ref[1] / deg], axis=1)
    h = (jnp.dot(mean, wl_ref[...], preferred_element_type=jnp.float32)
         + jnp.dot(x_ref[...], wr_ref[...],
                   preferred_element_type=jnp.float32) + b_ref[...])
    h_ref[...] = jnp.maximum(h, 0.0)
  grid = (N // BR,)
  return pl.pallas_call(
      body,
      grid=grid,
      in_specs=[
          pl.BlockSpec((NC, BR, HID), lambda i: (0, i, 0)),
          pl.BlockSpec((BR, CNTW), lambda i: (i, 0)),
          pl.BlockSpec((BR, IN_CH), lambda i: (i, 0)),
          pl.BlockSpec((IN_CH, HID), lambda i: (0, 0)),
          pl.BlockSpec((IN_CH, HID), lambda i: (0, 0)),
          pl.BlockSpec((HID,), lambda i: (0,)),
      ],
      out_specs=pl.BlockSpec((BR, HID), lambda i: (i, 0)),
      out_shape=jax.ShapeDtypeStruct((N, HID), jnp.float32),
  )(accA.reshape(NC, NPADR, HID), cnt, x, Wl, Wr, b)


def _tc_layer2(accp, cnt, h_prev, Wl, Wr, b):
  def body(acc_ref, cnt_ref, hp_ref, wl_ref, wr_ref, b_ref, h_ref):
    cntv = cnt_ref[...]
    deg = jnp.maximum(jnp.max(cntv, axis=1, keepdims=True), 1.0)
    mean = (acc_ref[0] + acc_ref[1]) / deg
    h = (jnp.dot(mean, wl_ref[...], preferred_element_type=jnp.float32)
         + jnp.dot(hp_ref[...], wr_ref[...],
                   preferred_element_type=jnp.float32) + b_ref[...])
    h_ref[...] = jnp.maximum(h, 0.0)
  grid = (N // BR,)
  return pl.pallas_call(
      body,
      grid=grid,
      in_specs=[
          pl.BlockSpec((NC, BR, HID), lambda i: (0, i, 0)),
          pl.BlockSpec((BR, CNTW), lambda i: (i, 0)),
          pl.BlockSpec((BR, HID), lambda i: (i, 0)),
          pl.BlockSpec((HID, HID), lambda i: (0, 0)),
          pl.BlockSpec((HID, HID), lambda i: (0, 0)),
          pl.BlockSpec((HID,), lambda i: (0,)),
      ],
      out_specs=pl.BlockSpec((BR, HID), lambda i: (i, 0)),
      out_shape=jax.ShapeDtypeStruct((N, HID), jnp.float32),
  )(accp.reshape(NC, NPADR, HID), cnt, h_prev, Wl, Wr, b)


def _tc_final(accp, cnt, h_prev, batch2d, Wl, Wr, b, Wc1, bc1, Wc2, bc2):
  def body(acc_ref, cnt_ref, hp_ref, b2d_ref, wl_ref, wr_ref, b_ref,
           wc1_ref, bc1_ref, wc2_ref, bc2_ref, out_ref):
    deg = _deg(cnt_ref)
    mean = (acc_ref[pl.ds(0, N)] + acc_ref[pl.ds(NPADR, N)]) / deg
    h = (jnp.dot(mean, wl_ref[...], preferred_element_type=jnp.float32)
         + jnp.dot(hp_ref[...], wr_ref[...],
                   preferred_element_type=jnp.float32) + b_ref[...])
    gid = b2d_ref[...]
    onehot = (gid == lax.broadcasted_iota(jnp.int32, (1, NG), 1))
    onehot = onehot.astype(jnp.float32)
    g = lax.dot_general(onehot, h, (((0,), (0,)), ((), ())),
                        preferred_element_type=jnp.float32,
                        precision=lax.Precision.HIGHEST)
    g = jnp.maximum(
        jnp.dot(g, wc1_ref[...], preferred_element_type=jnp.float32)
        + bc1_ref[...], 0.0)
    out_ref[...] = jnp.dot(
        g, wc2_ref[...], preferred_element_type=jnp.float32) + bc2_ref[...]
  outs = jax.ShapeDtypeStruct((NG, 1), jnp.float32)
  return pl.pallas_call(body, out_shape=outs)(
      accp, cnt, h_prev, batch2d, Wl, Wr, b, Wc1, bc1, Wc2, bc2)


def kernel(x, edge_index, batch, Wl1, Wr1, b1, Wl2, Wr2, b2, Wl3, Wr3, b3,
           Wc1, bc1, Wc2, bc2):
  src = edge_index[0].astype(jnp.int32)
  dst = edge_index[1].astype(jnp.int32)

  srcA = jnp.concatenate([src, jnp.zeros((PADA,), jnp.int32)])
  dstA = jnp.concatenate([dst, _pad_dst(PADA)])
  srcA = srcA.reshape(NS, 2, KH, CH)
  dstA = dstA.reshape(NS, 2, KH, CH)

  padsrc = jnp.zeros((NW, PAD), jnp.int32)
  paddst = jnp.broadcast_to(_pad_dst(PAD), (NW, PAD))
  srcs = jnp.concatenate([src.reshape(NW, EPW), padsrc], 1).reshape(NW, K, CH)
  dsts = jnp.concatenate([dst.reshape(NW, EPW), paddst], 1).reshape(NW, K, CH)

  xpad = jnp.concatenate([x, jnp.zeros((NPADR - N, IN_CH), jnp.float32)])
  xs = jnp.stack([xpad[:, :HID], xpad[:, HID:]])

  zrow = jnp.zeros((RPT, HID), jnp.float32)
  zcnt = jnp.zeros((RPT, CNTW), jnp.float32)
  ones = jnp.ones((CH, CNTW), jnp.float32)
  batch2d = batch.astype(jnp.int32).reshape(N, 1)

  zpadrows = jnp.zeros((NPADR - N, HID), jnp.float32)

  accA, cnt = _sc_pass_alledges()(xs, srcA, dstA, zrow, zcnt, ones)
  h1 = _tc_layer1(accA, cnt, x, Wl1, Wr1, b1)
  h1p = jnp.concatenate([h1, zpadrows])
  (acc2,) = _sc_pass_split()(h1p, srcs, dsts, zrow)
  h2 = _tc_layer2(acc2, cnt, h1, Wl2, Wr2, b2)
  h2p = jnp.concatenate([h2, zpadrows])
  (acc3,) = _sc_pass_split()(h2p, srcs, dsts, zrow)
  return _tc_final(acc3, cnt, h2, batch2d, Wl3, Wr3, b3, Wc1, bc1, Wc2, bc2)


# speedup vs baseline: 1.2167x; 1.0957x over previous
import jax
import jax.numpy as jnp
from jax import lax
from jax.experimental import pallas as pl
from jax.experimental.pallas import tpu as pltpu
from jax.experimental.pallas import tpu_sc as plsc

N = 10000
E = 320000
IN_CH = 128
HID = 64
NG = 64

NC, NS, L = 2, 16, 16
NW = NC * NS
CH = 128

RPT = 632
NPADR = RPT * NS
ZRT = NPADR // NS
CNTW = 16

EPW = E // NW
K = -(-EPW // CH)
PAD = K * CH - EPW

NSW = 4
EPT = E // NS
KH = -(-EPT // (NSW * CH))
PADA = NS * NSW * KH * CH - E


def _pad_dst(n_pad):
  return N + (jnp.arange(n_pad, dtype=jnp.int32) % (NPADR - N))


def _sc_pass_alledges():
  out_type = [jax.ShapeDtypeStruct((NC * NPADR, HID), jnp.float32),
              jax.ShapeDtypeStruct((NPADR, CNTW), jnp.float32)]
  scratch = {
      "srcv": pltpu.VMEM((KH, CH), jnp.int32),
      "dstv": pltpu.VMEM((KH, CH), jnp.int32),
      "buf": pltpu.VMEM((2, CH, HID), jnp.float32),
      "onesv": pltpu.VMEM((CH, CNTW), jnp.float32),
      "acc": pltpu.VMEM_SHARED((NPADR, HID), jnp.float32),
      "cntacc": pltpu.VMEM_SHARED((NPADR, CNTW), jnp.float32),
      "zsp": pltpu.VMEM_SHARED((NPADR, HID), jnp.float32),
      "sem": pltpu.SemaphoreType.DMA,
  }
  mesh = plsc.VectorSubcoreMesh(
      core_axis_name="c", subcore_axis_name="s",
      num_cores=NC, num_subcores=NS)

  def body(xs, srcs, dsts, zrow, zcnt, ones, out, cntout, *, srcv, dstv,
           buf, onesv, acc, cntacc, zsp, sem):
    cid = lax.axis_index("c")
    sid = lax.axis_index("s")
    pltpu.sync_copy(xs.at[cid, pl.ds(sid * ZRT, ZRT)],
                    zsp.at[pl.ds(sid * ZRT, ZRT)])
    pltpu.sync_copy(zrow, acc.at[pl.ds(sid * RPT, RPT)])
    pltpu.sync_copy(ones, onesv)

    @pl.when(cid == 0)
    def _():
      pltpu.sync_copy(zcnt, cntacc.at[pl.ds(sid * RPT, RPT)])
    plsc.subcore_barrier()

    for s in range(NSW):
      pltpu.sync_copy(srcs.at[sid, s], srcv)
      pltpu.sync_copy(dsts.at[sid, s], dstv)
      pltpu.async_copy(zsp.at[srcv.at[0]], buf.at[0], sem)

      def step(j, carry):
        p = lax.rem(j, 2)

        @pl.when(j + 1 < KH)
        def _():
          pltpu.async_copy(zsp.at[srcv.at[j + 1]], buf.at[1 - p], sem)
        pltpu.make_async_copy(zsp.at[srcv.at[j]], buf.at[p], sem).wait()
        pltpu.sync_copy(buf.at[p], acc.at[dstv.at[j]], add=True)

        @pl.when(cid == 0)
        def _():
          pltpu.sync_copy(onesv, cntacc.at[dstv.at[j]], add=True)
        return carry
      lax.fori_loop(0, KH, step, 0)

    plsc.subcore_barrier()
    obase = cid * NPADR + sid * RPT
    pltpu.sync_copy(acc.at[pl.ds(sid * RPT, RPT)], out.at[pl.ds(obase, RPT)])

    @pl.when(cid == 0)
    def _():
      pltpu.sync_copy(cntacc.at[pl.ds(sid * RPT, RPT)],
                      cntout.at[pl.ds(sid * RPT, RPT)])

  return pl.kernel(body, out_type=out_type, mesh=mesh,
                   scratch_types=scratch,
                   compiler_params=pltpu.CompilerParams(
                       use_tc_tiling_on_sc=False))


def _sc_pass_split():
  out_type = [jax.ShapeDtypeStruct((NC * NPADR, HID), jnp.float32)]
  scratch = {
      "srcv": pltpu.VMEM((K, CH), jnp.int32),
      "dstv": pltpu.VMEM((K, CH), jnp.int32),
      "buf": pltpu.VMEM((2, CH, HID), jnp.float32),
      "acc": pltpu.VMEM_SHARED((NPADR, HID), jnp.float32),
      "zsp": pltpu.VMEM_SHARED((NPADR, HID), jnp.float32),
      "sem": pltpu.SemaphoreType.DMA,
  }
  mesh = plsc.VectorSubcoreMesh(
      core_axis_name="c", subcore_axis_name="s",
      num_cores=NC, num_subcores=NS)

  def body(z, srcs, dsts, zrow, out, *, srcv, dstv, buf, acc, zsp, sem):
    cid = lax.axis_index("c")
    sid = lax.axis_index("s")
    wid = cid * NS + sid
    pltpu.sync_copy(srcs.at[wid], srcv)
    pltpu.sync_copy(dsts.at[wid], dstv)
    pltpu.sync_copy(z.at[pl.ds(sid * ZRT, ZRT)],
                    zsp.at[pl.ds(sid * ZRT, ZRT)])
    pltpu.sync_copy(zrow, acc.at[pl.ds(sid * RPT, RPT)])
    plsc.subcore_barrier()

    pltpu.async_copy(zsp.at[srcv.at[0]], buf.at[0], sem)

    def step(j, carry):
      p = lax.rem(j, 2)

      @pl.when(j + 1 < K)
      def _():
        pltpu.async_copy(zsp.at[srcv.at[j + 1]], buf.at[1 - p], sem)
      pltpu.make_async_copy(zsp.at[srcv.at[j]], buf.at[p], sem).wait()
      pltpu.sync_copy(buf.at[p], acc.at[dstv.at[j]], add=True)
      return carry
    lax.fori_loop(0, K, step, 0)

    plsc.subcore_barrier()
    obase = cid * NPADR + sid * RPT
    pltpu.sync_copy(acc.at[pl.ds(sid * RPT, RPT)], out.at[pl.ds(obase, RPT)])

  return pl.kernel(body, out_type=out_type, mesh=mesh,
                   scratch_types=scratch,
                   compiler_params=pltpu.CompilerParams(
                       use_tc_tiling_on_sc=False))


def _deg(cnt_ref):
  cnt = cnt_ref[pl.ds(0, N)]
  return jnp.maximum(jnp.max(cnt, axis=1, keepdims=True), 1.0)


BR = 2000


def _tc_layer1(accA, cnt, x, Wl, Wr, b):
  def body(acc_ref, cnt_ref, x_ref, wl_ref, wr_ref, b_ref, h_ref):
    cntv = cnt_ref[...]
    deg = jnp.maximum(jnp.max(cntv, axis=1, keepdims=True), 1.0)
    mean = jnp.concatenate([acc_ref[0] / deg, acc_ref[1] / deg], axis=1)
    h = (jnp.dot(mean, wl_ref[...], preferred_element_type=jnp.float32)
         + jnp.dot(x_ref[...], wr_ref[...],
                   preferred_element_type=jnp.float32) + b_ref[...])
    h_ref[...] = jnp.maximum(h, 0.0)
  grid = (N // BR,)
  return pl.pallas_call(
      body,
      grid=grid,
      in_specs=[
          pl.BlockSpec((NC, BR, HID), lambda i: (0, i, 0)),
          pl.BlockSpec((BR, CNTW), lambda i: (i, 0)),
          pl.BlockSpec((BR, IN_CH), lambda i: (i, 0)),
          pl.BlockSpec((IN_CH, HID), lambda i: (0, 0)),
          pl.BlockSpec((IN_CH, HID), lambda i: (0, 0)),
          pl.BlockSpec((HID,), lambda i: (0,)),
      ],
      out_specs=pl.BlockSpec((BR, HID), lambda i: (i, 0)),
      out_shape=jax.ShapeDtypeStruct((N, HID), jnp.float32),
  )(accA.reshape(NC, NPADR, HID), cnt, x, Wl, Wr, b)


def _tc_layer2(accp, cnt, h_prev, Wl, Wr, b):
  def body(acc_ref, cnt_ref, hp_ref, wl_ref, wr_ref, b_ref, h_ref):
    cntv = cnt_ref[...]
    deg = jnp.maximum(jnp.max(cntv, axis=1, keepdims=True), 1.0)
    mean = (acc_ref[0] + acc_ref[1]) / deg
    h = (jnp.dot(mean, wl_ref[...], preferred_element_type=jnp.float32)
         + jnp.dot(hp_ref[...], wr_ref[...],
                   preferred_element_type=jnp.float32) + b_ref[...])
    h_ref[...] = jnp.maximum(h, 0.0)
  grid = (N // BR,)
  return pl.pallas_call(
      body,
      grid=grid,
      in_specs=[
          pl.BlockSpec((NC, BR, HID), lambda i: (0, i, 0)),
          pl.BlockSpec((BR, CNTW), lambda i: (i, 0)),
          pl.BlockSpec((BR, HID), lambda i: (i, 0)),
          pl.BlockSpec((HID, HID), lambda i: (0, 0)),
          pl.BlockSpec((HID, HID), lambda i: (0, 0)),
          pl.BlockSpec((HID,), lambda i: (0,)),
      ],
      out_specs=pl.BlockSpec((BR, HID), lambda i: (i, 0)),
      out_shape=jax.ShapeDtypeStruct((N, HID), jnp.float32),
  )(accp.reshape(NC, NPADR, HID), cnt, h_prev, Wl, Wr, b)


def _tc_final(accp, cnt, h_prev, batch2d, Wl, Wr, b, Wc1, bc1, Wc2, bc2):
  def body(acc_ref, cnt_ref, hp_ref, b2d_ref, wl_ref, wr_ref, b_ref,
           wc1_ref, bc1_ref, wc2_ref, bc2_ref, out_ref):
    deg = _deg(cnt_ref)
    mean = (acc_ref[pl.ds(0, N)] + acc_ref[pl.ds(NPADR, N)]) / deg
    h = (jnp.dot(mean, wl_ref[...], preferred_element_type=jnp.float32)
         + jnp.dot(hp_ref[...], wr_ref[...],
                   preferred_element_type=jnp.float32) + b_ref[...])
    gid = b2d_ref[...]
    onehot = (gid == lax.broadcasted_iota(jnp.int32, (1, NG), 1))
    onehot = onehot.astype(jnp.float32)
    g = lax.dot_general(onehot, h, (((0,), (0,)), ((), ())),
                        preferred_element_type=jnp.float32,
                        precision=lax.Precision.HIGHEST)
    g = jnp.maximum(
        jnp.dot(g, wc1_ref[...], preferred_element_type=jnp.float32)
        + bc1_ref[...], 0.0)
    out_ref[...] = jnp.dot(
        g, wc2_ref[...], preferred_element_type=jnp.float32) + bc2_ref[...]
  outs = jax.ShapeDtypeStruct((NG, 1), jnp.float32)
  return pl.pallas_call(body, out_shape=outs)(
      accp, cnt, h_prev, batch2d, Wl, Wr, b, Wc1, bc1, Wc2, bc2)


def kernel(x, edge_index, batch, Wl1, Wr1, b1, Wl2, Wr2, b2, Wl3, Wr3, b3,
           Wc1, bc1, Wc2, bc2):
  src = edge_index[0].astype(jnp.int32)
  dst = edge_index[1].astype(jnp.int32)

  srcA = jnp.concatenate([src, jnp.zeros((PADA,), jnp.int32)])
  dstA = jnp.concatenate([dst, _pad_dst(PADA)])
  srcA = srcA.reshape(NS, NSW, KH, CH)
  dstA = dstA.reshape(NS, NSW, KH, CH)

  padsrc = jnp.zeros((NW, PAD), jnp.int32)
  paddst = jnp.broadcast_to(_pad_dst(PAD), (NW, PAD))
  srcs = jnp.concatenate([src.reshape(NW, EPW), padsrc], 1).reshape(NW, K, CH)
  dsts = jnp.concatenate([dst.reshape(NW, EPW), paddst], 1).reshape(NW, K, CH)

  xpad = jnp.concatenate([x, jnp.zeros((NPADR - N, IN_CH), jnp.float32)])
  xs = jnp.stack([xpad[:, :HID], xpad[:, HID:]])

  zrow = jnp.zeros((RPT, HID), jnp.float32)
  zcnt = jnp.zeros((RPT, CNTW), jnp.float32)
  ones = jnp.ones((CH, CNTW), jnp.float32)
  batch2d = batch.astype(jnp.int32).reshape(N, 1)

  zpadrows = jnp.zeros((NPADR - N, HID), jnp.float32)

  accA, cnt = _sc_pass_alledges()(xs, srcA, dstA, zrow, zcnt, ones)
  h1 = _tc_layer1(accA, cnt, x, Wl1, Wr1, b1)
  h1p = jnp.concatenate([h1, zpadrows])
  (acc2,) = _sc_pass_split()(h1p, srcs, dsts, zrow)
  h2 = _tc_layer2(acc2, cnt, h1, Wl2, Wr2, b2)
  h2p = jnp.concatenate([h2, zpadrows])
  (acc3,) = _sc_pass_split()(h2p, srcs, dsts, zrow)
  return _tc_final(acc3, cnt, h2, batch2d, Wl3, Wr3, b3, Wc1, bc1, Wc2, bc2)
